# baseline (speedup 1.0000x reference)
import functools

import jax
import jax.numpy as jnp
from jax import lax
from jax.experimental import pallas as pl
from jax.experimental.pallas import tpu as pltpu
from jax.experimental.pallas import tpu_sc as plsc

_NW = 32
_CHUNK = 4


def _sc_mean_body(nv_hbm, mk_hbm, out_hbm, nvbuf, mkbuf, obuf):
    n_rows = nv_hbm.shape[0]
    n = nv_hbm.shape[1]
    d = nv_hbm.shape[2]
    rpw = n_rows // _NW
    wid = lax.axis_index("s") * 2 + lax.axis_index("c")
    base = wid * rpw
    inv_n = 1.0 / n

    def chunk_body(g, carry):
        row0 = base + g * _CHUNK
        pltpu.sync_copy(nv_hbm.at[pl.ds(row0, _CHUNK)], nvbuf)
        pltpu.sync_copy(mk_hbm.at[pl.ds(row0, _CHUNK)], mkbuf)
        for r in range(_CHUNK):
            mrow = mkbuf[r]
            for j in range(d // 16):
                acc = mrow[0] * nvbuf[r, 0, pl.ds(j * 16, 16)]
                for t in range(1, n):
                    acc = acc + mrow[t] * nvbuf[r, t, pl.ds(j * 16, 16)]
                obuf[r, pl.ds(j * 16, 16)] = acc * inv_n
        pltpu.sync_copy(obuf, out_hbm.at[pl.ds(row0, _CHUNK)])
        return carry

    lax.fori_loop(0, rpw // _CHUNK, chunk_body, 0)


def _tc_body(self_ref, emb_ref, mean_ref, w_ref, b_ref, out_ref):
    w = w_ref[...]
    d = self_ref.shape[1]
    acc = jnp.dot(self_ref[...], w[0:d], preferred_element_type=jnp.float32)
    acc = acc + jnp.dot(mean_ref[...], w[d:2 * d],
                        preferred_element_type=jnp.float32)
    acc = acc + jnp.dot(emb_ref[...], w[2 * d:3 * d],
                        preferred_element_type=jnp.float32)
    out_ref[...] = jnp.maximum(acc + b_ref[...], 0.0)


def kernel(self_vectors, neighbor_vectors, masks, node_emb, W, b):
    B_, _, H_, D_ = self_vectors.shape
    N_ = neighbor_vectors.shape[2]
    O_ = W.shape[1]
    BH = B_ * H_
    sv = self_vectors.reshape(BH, D_)
    nv3 = neighbor_vectors.reshape(BH, N_, D_)
    mk = masks.reshape(BH, N_)
    ne = node_emb.reshape(BH, D_)
    b2 = b.reshape(1, O_)

    mesh = plsc.VectorSubcoreMesh(core_axis_name="c", subcore_axis_name="s")
    sc_mean = pl.kernel(
        _sc_mean_body,
        out_type=jax.ShapeDtypeStruct((BH, D_), jnp.float32),
        mesh=mesh,
        scratch_types=[
            pltpu.VMEM((_CHUNK, N_, D_), jnp.float32),
            pltpu.VMEM((_CHUNK, N_), jnp.float32),
            pltpu.VMEM((_CHUNK, D_), jnp.float32),
        ],
    )
    mean = sc_mean(nv3, mk)

    R = 2048
    grid = (BH // R,)
    out = pl.pallas_call(
        _tc_body,
        grid=grid,
        in_specs=[
            pl.BlockSpec((R, D_), lambda i: (i, 0)),
            pl.BlockSpec((R, D_), lambda i: (i, 0)),
            pl.BlockSpec((R, D_), lambda i: (i, 0)),
            pl.BlockSpec((3 * D_, O_), lambda i: (0, 0)),
            pl.BlockSpec((1, O_), lambda i: (0, 0)),
        ],
        out_specs=pl.BlockSpec((R, O_), lambda i: (i, 0)),
        out_shape=jax.ShapeDtypeStruct((BH, O_), jnp.float32),
    )(sv, ne, mean, W, b2)
    return out.reshape(B_, 1, H_, O_)


# speedup vs baseline: 3.7773x; 3.7773x over previous
import jax
import jax.numpy as jnp
from jax import lax
from jax.experimental import pallas as pl
from jax.experimental.pallas import tpu as pltpu
from jax.experimental.pallas import tpu_sc as plsc

_NW = 32
_CHUNK = 4
_F = 2048


def _sc_mean_body(nv_hbm, mk_hbm, out_hbm, nvbuf, mkbuf, obuf):
    n_rows = _F
    n = nv_hbm.shape[1]
    d = nv_hbm.shape[2]
    rpw = n_rows // _NW
    wid = lax.axis_index("s") * 2 + lax.axis_index("c")
    base = wid * rpw
    inv_n = 1.0 / n

    def chunk_body(g, carry):
        row0 = base + g * _CHUNK
        pltpu.sync_copy(nv_hbm.at[pl.ds(row0, _CHUNK)], nvbuf)
        pltpu.sync_copy(mk_hbm.at[pl.ds(row0, _CHUNK)], mkbuf)
        for r in range(_CHUNK):
            mrow = mkbuf[r]
            for j in range(d // 16):
                acc = mrow[0] * nvbuf[r, 0, pl.ds(j * 16, 16)]
                for t in range(1, n):
                    acc = acc + mrow[t] * nvbuf[r, t, pl.ds(j * 16, 16)]
                obuf[r, pl.ds(j * 16, 16)] = acc * inv_n
        pltpu.sync_copy(obuf, out_hbm.at[pl.ds(row0, _CHUNK)])
        return carry

    lax.fori_loop(0, rpw // _CHUNK, chunk_body, 0)


def _tc_fused_body(self_ref, emb_ref, mask_ref, neigh_ref, w_ref, b_ref,
                   out_ref):
    nv = neigh_ref[...]
    m = mask_ref[...] * (1.0 / nv.shape[1])
    mean = jnp.sum(nv * m[:, :, None], axis=1)
    w = w_ref[...]
    d = mean.shape[1]
    acc = jnp.dot(self_ref[...], w[0:d], preferred_element_type=jnp.float32)
    acc = acc + jnp.dot(mean, w[d:2 * d], preferred_element_type=jnp.float32)
    acc = acc + jnp.dot(emb_ref[...], w[2 * d:3 * d],
                        preferred_element_type=jnp.float32)
    out_ref[...] = jnp.maximum(acc + b_ref[...], 0.0)


def _tc_mean_body(self_ref, emb_ref, mean_ref, w_ref, b_ref, alias_ref,
                  out_ref):
    del alias_ref
    w = w_ref[...]
    d = self_ref.shape[1]
    acc = jnp.dot(self_ref[...], w[0:d], preferred_element_type=jnp.float32)
    acc = acc + jnp.dot(mean_ref[...], w[d:2 * d],
                        preferred_element_type=jnp.float32)
    acc = acc + jnp.dot(emb_ref[...], w[2 * d:3 * d],
                        preferred_element_type=jnp.float32)
    out_ref[...] = jnp.maximum(acc + b_ref[...], 0.0)


def kernel(self_vectors, neighbor_vectors, masks, node_emb, W, b):
    B_, _, H_, D_ = self_vectors.shape
    N_ = neighbor_vectors.shape[2]
    O_ = W.shape[1]
    BH = B_ * H_
    sv = self_vectors.reshape(BH, D_)
    nv3 = neighbor_vectors.reshape(BH, N_, D_)
    mk = masks.reshape(BH, N_)
    ne = node_emb.reshape(BH, D_)
    b2 = b.reshape(1, O_)

    mesh = plsc.VectorSubcoreMesh(core_axis_name="c", subcore_axis_name="s")
    sc_mean = pl.kernel(
        _sc_mean_body,
        out_type=jax.ShapeDtypeStruct((_F, D_), jnp.float32),
        mesh=mesh,
        scratch_types=[
            pltpu.VMEM((_CHUNK, N_, D_), jnp.float32),
            pltpu.VMEM((_CHUNK, N_), jnp.float32),
            pltpu.VMEM((_CHUNK, D_), jnp.float32),
        ],
    )
    mean_sc = sc_mean(nv3, mk)

    R = 2048
    off = _F // R
    grid_a = ((BH - _F) // R,)
    out_a = pl.pallas_call(
        _tc_fused_body,
        grid=grid_a,
        in_specs=[
            pl.BlockSpec((R, D_), lambda i: (i + off, 0)),
            pl.BlockSpec((R, D_), lambda i: (i + off, 0)),
            pl.BlockSpec((R, N_), lambda i: (i + off, 0)),
            pl.BlockSpec((R, N_, D_), lambda i: (i + off, 0, 0)),
            pl.BlockSpec((3 * D_, O_), lambda i: (0, 0)),
            pl.BlockSpec((1, O_), lambda i: (0, 0)),
        ],
        out_specs=pl.BlockSpec((R, O_), lambda i: (i + off, 0)),
        out_shape=jax.ShapeDtypeStruct((BH, O_), jnp.float32),
    )(sv, ne, mk, nv3, W, b2)

    grid_b = (_F // R,)
    out = pl.pallas_call(
        _tc_mean_body,
        grid=grid_b,
        in_specs=[
            pl.BlockSpec((R, D_), lambda i: (i, 0)),
            pl.BlockSpec((R, D_), lambda i: (i, 0)),
            pl.BlockSpec((R, D_), lambda i: (i, 0)),
            pl.BlockSpec((3 * D_, O_), lambda i: (0, 0)),
            pl.BlockSpec((1, O_), lambda i: (0, 0)),
            pl.BlockSpec(memory_space=pl.ANY),
        ],
        out_specs=pl.BlockSpec((R, O_), lambda i: (i, 0)),
        out_shape=jax.ShapeDtypeStruct((BH, O_), jnp.float32),
        input_output_aliases={5: 0},
    )(sv, ne, mean_sc, W, b2, out_a)
    return out.reshape(B_, 1, H_, O_)


# speedup vs baseline: 4.7918x; 1.2686x over previous
import jax
import jax.numpy as jnp
from jax.experimental import pallas as pl


def _agg_body(self_ref, emb_ref, mask_ref, neigh_ref, w_ref, b_ref, out_ref):
    nv = neigh_ref[...]
    m = mask_ref[...] * (1.0 / nv.shape[1])
    mean = jnp.sum(nv * m[:, :, None], axis=1)
    w = w_ref[...]
    d = mean.shape[1]
    acc = jnp.dot(self_ref[...], w[0:d], preferred_element_type=jnp.float32)
    acc = acc + jnp.dot(mean, w[d:2 * d], preferred_element_type=jnp.float32)
    acc = acc + jnp.dot(emb_ref[...], w[2 * d:3 * d],
                        preferred_element_type=jnp.float32)
    out_ref[...] = jnp.maximum(acc + b_ref[...], 0.0)


def kernel(self_vectors, neighbor_vectors, masks, node_emb, W, b):
    B_, _, H_, D_ = self_vectors.shape
    N_ = neighbor_vectors.shape[2]
    O_ = W.shape[1]
    BH = B_ * H_
    sv = self_vectors.reshape(BH, D_)
    nv = neighbor_vectors.reshape(BH, N_, D_)
    mk = masks.reshape(BH, N_)
    ne = node_emb.reshape(BH, D_)
    b2 = b.reshape(1, O_)

    R = 2048
    grid = (BH // R,)
    out = pl.pallas_call(
        _agg_body,
        grid=grid,
        in_specs=[
            pl.BlockSpec((R, D_), lambda i: (i, 0)),
            pl.BlockSpec((R, D_), lambda i: (i, 0)),
            pl.BlockSpec((R, N_), lambda i: (i, 0)),
            pl.BlockSpec((R, N_, D_), lambda i: (i, 0, 0)),
            pl.BlockSpec((3 * D_, O_), lambda i: (0, 0)),
            pl.BlockSpec((1, O_), lambda i: (0, 0)),
        ],
        out_specs=pl.BlockSpec((R, O_), lambda i: (i, 0)),
        out_shape=jax.ShapeDtypeStruct((BH, O_), jnp.float32),
    )(sv, ne, mk, nv, W, b2)
    return out.reshape(B_, 1, H_, O_)
